# baseline (speedup 1.0000x reference)
import functools

import jax
import jax.numpy as jnp
from jax import lax
from jax.experimental import pallas as pl
from jax.experimental.pallas import tpu as pltpu
from jax.experimental.pallas import tpu_sc as plsc

NUM_FEATURES = 128
NUM_EDGES = 160000
NUM_NODES = 10000
ROWS = NUM_NODES + NUM_EDGES
BATCH = 4
TOTAL_ROWS = BATCH * ROWS

NC, NS = 2, 16
N_IDX = 128

CHUNK = 400
CHUNKS_PER_BATCH = ROWS // CHUNK
NODE_CHUNK0 = NUM_EDGES // CHUNK
K_MAX = -(-CHUNKS_PER_BATCH // NS)
K_NODE0 = NODE_CHUNK0 // NS

_sc_mesh = plsc.VectorSubcoreMesh(core_axis_name="c", subcore_axis_name="s")


@functools.partial(
    pl.kernel,
    out_type=jax.ShapeDtypeStruct((TOTAL_ROWS, NUM_FEATURES), jnp.float32),
    mesh=_sc_mesh,
    scratch_types=[
        pltpu.VMEM((CHUNK, NUM_FEATURES), jnp.float32),
        pltpu.VMEM((N_IDX,), jnp.int32),
        pltpu.VMEM((N_IDX, NUM_FEATURES), jnp.float32),
        pltpu.SemaphoreType.DMA,
        pltpu.SemaphoreType.DMA,
    ],
)
def _sc_all(x_hbm, idx_hbm, out_ref, zbuf, idx_v, rows_v, sem_z, sem_s):
    c = lax.axis_index("c")
    s = lax.axis_index("s")
    is_scatterer = s >= NS - 2
    b_sc = jnp.where(s == NS - 1, c, c + 2)

    @pl.when(is_scatterer)
    def _():
        pltpu.async_copy(idx_hbm.at[0], idx_v, sem_s)
        pltpu.async_copy(x_hbm.at[pl.ds(b_sc * N_IDX, N_IDX)], rows_v, sem_s)

    z16 = jnp.zeros((16,), jnp.float32)

    @pl.loop(0, CHUNK)
    def _(i):
        for j in range(NUM_FEATURES // 16):
            zbuf[i, pl.ds(j * 16, 16)] = z16

    def chunk_dst(b2, k):
        batch = c + 2 * b2
        r = k * NS + s
        return r, out_ref.at[pl.ds(batch * ROWS + r * CHUNK, CHUNK)]

    node_ks = range(K_NODE0, K_MAX)
    edge_ks = range(K_NODE0)

    for b2 in range(2):
        for k in node_ks:
            r, dst = chunk_dst(b2, k)

            @pl.when(r < CHUNKS_PER_BATCH)
            def _():
                pltpu.async_copy(zbuf, dst, sem_z)

    for b2 in range(2):
        for k in node_ks:
            r, dst = chunk_dst(b2, k)

            @pl.when(r < CHUNKS_PER_BATCH)
            def _():
                pltpu.make_async_copy(zbuf, dst, sem_z).wait()

    plsc.subcore_barrier()

    @pl.when(is_scatterer)
    def _():
        pltpu.make_async_copy(idx_hbm.at[0], idx_v, sem_s).wait()
        pltpu.make_async_copy(
            x_hbm.at[pl.ds(b_sc * N_IDX, N_IDX)], rows_v, sem_s
        ).wait()
        off = b_sc * ROWS + NUM_EDGES
        for j in range(N_IDX // 16):
            sl = pl.ds(j * 16, 16)
            idx_v[sl] = idx_v[sl] + off
        pltpu.sync_copy(rows_v, out_ref.at[idx_v])

    for b2 in range(2):
        for k in edge_ks:
            _, dst = chunk_dst(b2, k)
            pltpu.async_copy(zbuf, dst, sem_z)

    for b2 in range(2):
        for k in edge_ks:
            _, dst = chunk_dst(b2, k)
            pltpu.make_async_copy(zbuf, dst, sem_z).wait()


def kernel(x, pivotal_nodes):
    bsz, n_idx, f = x.shape
    x_flat = x.reshape(bsz * n_idx, f)
    idx2 = pivotal_nodes.reshape(1, N_IDX)
    return _sc_all(x_flat, idx2).reshape(bsz, ROWS, f)


# speedup vs baseline: 1.0279x; 1.0279x over previous
import functools

import jax
import jax.numpy as jnp
from jax import lax
from jax.experimental import pallas as pl
from jax.experimental.pallas import tpu as pltpu
from jax.experimental.pallas import tpu_sc as plsc

NUM_FEATURES = 128
NUM_EDGES = 160000
NUM_NODES = 10000
ROWS = NUM_NODES + NUM_EDGES
BATCH = 4
TOTAL_ROWS = BATCH * ROWS
FILL_BLOCK = 17000

NC, NS = 2, 16
NW = NC * NS
N_IDX = 128
ROWS_PER_W = BATCH * N_IDX // NW
IDX_GROUPS = N_IDX // ROWS_PER_W


def _fill_body(out_ref):
    out_ref[...] = jnp.zeros_like(out_ref)


def _tc_fill():
    return pl.pallas_call(
        _fill_body,
        grid=(TOTAL_ROWS // FILL_BLOCK,),
        out_specs=pl.BlockSpec((FILL_BLOCK, NUM_FEATURES), lambda i: (i, 0)),
        out_shape=jax.ShapeDtypeStruct((TOTAL_ROWS, NUM_FEATURES), jnp.float32),
    )()


_sc_mesh = plsc.VectorSubcoreMesh(core_axis_name="c", subcore_axis_name="s")


@functools.partial(
    pl.kernel,
    out_type=(),
    mesh=_sc_mesh,
    scratch_types=[
        pltpu.VMEM((ROWS_PER_W,), jnp.int32),
        pltpu.VMEM((ROWS_PER_W, NUM_FEATURES), jnp.float32),
    ],
)
def _sc_scatter(out_ref, x_hbm, idx_hbm, idx_v, rows_v):
    wid = lax.axis_index("s") * NC + lax.axis_index("c")
    b = wid // IDX_GROUPS
    g = wid % IDX_GROUPS
    pltpu.sync_copy(idx_hbm.at[g], idx_v)
    pltpu.sync_copy(x_hbm.at[pl.ds(wid * ROWS_PER_W, ROWS_PER_W)], rows_v)
    idx_v[...] = idx_v[...] + (b * ROWS + NUM_EDGES)
    pltpu.sync_copy(rows_v, out_ref.at[idx_v])


def kernel(x, pivotal_nodes):
    bsz, n_idx, f = x.shape
    x_flat = x.reshape(bsz * n_idx, f)
    idx2 = pivotal_nodes.reshape(IDX_GROUPS, ROWS_PER_W)
    out_ref = jax.new_ref(_tc_fill())
    _sc_scatter(out_ref, x_flat, idx2)
    return out_ref[...].reshape(bsz, ROWS, f)
